# baseline (speedup 1.0000x reference)
import jax
import jax.numpy as jnp
from jax import lax
from jax.experimental import pallas as pl
from jax.experimental.pallas import tpu as pltpu
from jax.experimental.pallas import tpu_sc as plsc

_TOT = 320
_B, _L = 4096, 50
_N = _B * _L
_C = 40
_NC, _NS = 2, 16
_NW = _NC * _NS
_PER_W = _N // _NW
_STEPS = _PER_W // _C


def _sc_body(i0, i1, i2, i3, i4, w0, w1, w2, w3, w4, out,
             iv0, iv1, iv2, iv3, iv4,
             c0, c1, b0, b1, f0, f1, r0, r1, p0, p1, a0, a1,
             sg0, sg1, sw0, sw1):
    wid = lax.axis_index("s") * _NC + lax.axis_index("c")
    tok0 = wid * _PER_W
    idx_hbm = (i0, i1, i2, i3, i4)
    idx_v = (iv0, iv1, iv2, iv3, iv4)
    tables = (w0, w1, w2, w3, w4)
    dsts = ((c0, b0, f0, r0, p0), (c1, b1, f1, r1, p1))
    asm = (a0, a1)
    semg = (sg0, sg1)
    semw = (sw0, sw1)

    for t in range(5):
        pltpu.sync_copy(idx_hbm[t].at[pl.ds(tok0, _PER_W)], idx_v[t])

    def fire_gathers(j, s):
        for t in range(5):
            pltpu.make_async_copy(
                tables[t].at[idx_v[t].at[pl.ds(j * _C, _C)]],
                dsts[s][t], semg[s]).start()

    def wait_gathers(s):
        for t in range(5):
            pltpu.make_async_copy(
                tables[t].at[pl.ds(0, _C)], dsts[s][t], semg[s]).wait()

    def wait_writes(s):
        pltpu.make_async_copy(
            dsts[s][0], out.at[pl.ds(0, _C), pl.ds(0, 128)], semw[s]).wait()
        pltpu.make_async_copy(
            asm[s], out.at[pl.ds(0, _C), pl.ds(128, 192)], semw[s]).wait()

    fire_gathers(0, 0)

    def step(j, carry):
        s = lax.rem(j, 2)

        @pl.when(s == 0)
        def _():
            wait_gathers(0)

        @pl.when(s == 1)
        def _():
            wait_gathers(1)

        @pl.when((j >= 1) & (s == 0))
        def _():
            wait_writes(1)

        @pl.when((j >= 1) & (s == 1))
        def _():
            wait_writes(0)

        @pl.when((j < _STEPS - 1) & (s == 0))
        def _():
            fire_gathers(j + 1, 1)

        @pl.when((j < _STEPS - 1) & (s == 1))
        def _():
            fire_gathers(j + 1, 0)

        def assemble(i, carry3, rb, rf, rr, rp, am):
            am[i, pl.ds(0, 16)] = rb[i, pl.ds(0, 16)]
            am[i, pl.ds(16, 16)] = rb[i, pl.ds(16, 16)]
            am[i, pl.ds(32, 16)] = rf[i, pl.ds(32, 16)]
            am[i, pl.ds(48, 16)] = rf[i, pl.ds(48, 16)]
            am[i, pl.ds(64, 16)] = rr[i, pl.ds(64, 16)]
            am[i, pl.ds(80, 16)] = rr[i, pl.ds(80, 16)]
            am[i, pl.ds(96, 16)] = rr[i, pl.ds(96, 16)]
            am[i, pl.ds(112, 16)] = rr[i, pl.ds(112, 16)]
            am[i, pl.ds(128, 16)] = rp[i, pl.ds(0, 16)]
            am[i, pl.ds(144, 16)] = rp[i, pl.ds(16, 16)]
            am[i, pl.ds(160, 16)] = rp[i, pl.ds(32, 16)]
            am[i, pl.ds(176, 16)] = rp[i, pl.ds(48, 16)]
            return carry3

        row = tok0 + j * _C

        def emit(s_const):
            rc, rb, rf, rr, rp = dsts[s_const]
            lax.fori_loop(
                0, _C,
                lambda i, c: assemble(i, c, rb, rf, rr, rp, asm[s_const]), 0)
            pltpu.make_async_copy(
                rc, out.at[pl.ds(row, _C), pl.ds(0, 128)],
                semw[s_const]).start()
            pltpu.make_async_copy(
                asm[s_const], out.at[pl.ds(row, _C), pl.ds(128, 192)],
                semw[s_const]).start()

        @pl.when(s == 0)
        def _():
            emit(0)

        @pl.when(s == 1)
        def _():
            emit(1)

        return carry

    lax.fori_loop(0, _STEPS, step, 0)
    wait_writes((_STEPS - 1) % 2)


def _pad_cols(w, col0):
    v, d = w.shape
    return jnp.zeros((v, 128), jnp.float32).at[:, col0:col0 + d].set(w)


def kernel(idx_char, idx_bound, idx_flag, idx_radical, idx_pinyin,
           W_char, W_bound, W_flag, W_radical, W_pinyin):
    idxs = [a.reshape(_N).astype(jnp.int32)
            for a in (idx_char, idx_bound, idx_flag, idx_radical, idx_pinyin)]
    tables = [W_char,
              _pad_cols(W_bound, 0),
              _pad_cols(W_flag, 32),
              _pad_cols(W_radical, 64),
              _pad_cols(W_pinyin, 0)]
    scratch = ([pltpu.VMEM((_PER_W,), jnp.int32) for _ in range(5)]
               + [pltpu.VMEM((_C, 128), jnp.float32) for _ in range(10)]
               + [pltpu.VMEM((_C, 192), jnp.float32) for _ in range(2)]
               + [pltpu.SemaphoreType.DMA for _ in range(4)])
    k = pl.kernel(
        _sc_body,
        out_type=jax.ShapeDtypeStruct((_N, _TOT), jnp.float32),
        mesh=plsc.VectorSubcoreMesh(core_axis_name="c", subcore_axis_name="s"),
        scratch_types=scratch,
    )
    out = k(*idxs, *tables)
    return out.reshape(_B, _L, _TOT)


# speedup vs baseline: 1.1249x; 1.1249x over previous
import jax
import jax.numpy as jnp
from jax import lax
from jax.experimental import pallas as pl
from jax.experimental.pallas import tpu as pltpu
from jax.experimental.pallas import tpu_sc as plsc

_TOT = 320
_B, _L = 4096, 50
_N = _B * _L
_C = 40
_NC, _NS = 2, 16
_NW = _NC * _NS
_PER_W = _N // _NW
_STEPS = _PER_W // _C


def _sc_body(ic, ib, if_, ir, ip, wc, wb, wf, wr, wp, out,
             ivc, ivb, ivf, ivr, ivp, ivr2, ivp2, tb, tf,
             c0, c1, r0, r1, p0, p1, a0, a1,
             sg0, sg1, sw0, sw1):
    wid = lax.axis_index("s") * _NC + lax.axis_index("c")
    tok0 = wid * _PER_W
    gsrc = (wc, wr, wp)
    gidx = (ivc, ivr2, ivp2)
    dsts = ((c0, r0, p0), (c1, r1, p1))
    asm = (a0, a1)
    semg = (sg0, sg1)
    semw = (sw0, sw1)

    for src, dst in ((ic, ivc), (ib, ivb), (if_, ivf), (ir, ivr), (ip, ivp)):
        pltpu.sync_copy(src.at[pl.ds(tok0, _PER_W)], dst)
    pltpu.sync_copy(wb, tb)
    pltpu.sync_copy(wf, tf)

    def halve(k, carry):
        s = pl.ds(k * 16, 16)
        ivr2[s] = lax.shift_right_logical(ivr[s], 1)
        ivp2[s] = lax.shift_right_logical(ivp[s], 1)
        return carry

    lax.fori_loop(0, _PER_W // 16, halve, 0)

    iota = lax.iota(jnp.int32, 16)

    def fire_gathers(j, s):
        for t in range(3):
            pltpu.make_async_copy(
                gsrc[t].at[gidx[t].at[pl.ds(j * _C, _C)]],
                dsts[s][t], semg[s]).start()

    def wait_gathers(s):
        for t in range(3):
            pltpu.make_async_copy(
                gsrc[t].at[pl.ds(0, _C)], dsts[s][t], semg[s]).wait()

    def wait_writes(s):
        pltpu.make_async_copy(
            dsts[s][0], out.at[pl.ds(0, _C), pl.ds(0, 128)], semw[s]).wait()
        pltpu.make_async_copy(
            asm[s], out.at[pl.ds(0, _C), pl.ds(128, 192)], semw[s]).wait()

    fire_gathers(0, 0)

    def step(j, carry):
        s = lax.rem(j, 2)

        @pl.when(s == 0)
        def _():
            wait_gathers(0)

        @pl.when(s == 1)
        def _():
            wait_gathers(1)

        @pl.when((j >= 1) & (s == 0))
        def _():
            wait_writes(1)

        @pl.when((j >= 1) & (s == 1))
        def _():
            wait_writes(0)

        @pl.when((j < _STEPS - 1) & (s == 0))
        def _():
            fire_gathers(j + 1, 1)

        @pl.when((j < _STEPS - 1) & (s == 1))
        def _():
            fire_gathers(j + 1, 0)

        def assemble(i, carry3, rr, rp, am):
            pos = jnp.full((16,), j * _C + i, jnp.int32)
            row = jnp.full((16,), i, jnp.int32)
            vb = plsc.load_gather(ivb, [pos]) * 32
            vf = plsc.load_gather(ivf, [pos]) * 32
            vr = plsc.load_gather(ivr, [pos])
            vp = plsc.load_gather(ivp, [pos])
            roff = (vr & 1) * 64
            poff = (vp & 1) * 64
            am[i, pl.ds(0, 16)] = plsc.load_gather(tb, [vb + iota])
            am[i, pl.ds(16, 16)] = plsc.load_gather(tb, [vb + (16 + iota)])
            am[i, pl.ds(32, 16)] = plsc.load_gather(tf, [vf + iota])
            am[i, pl.ds(48, 16)] = plsc.load_gather(tf, [vf + (16 + iota)])
            for c in range(4):
                am[i, pl.ds(64 + c * 16, 16)] = plsc.load_gather(
                    rr, [row, roff + (c * 16 + iota)])
            for c in range(4):
                am[i, pl.ds(128 + c * 16, 16)] = plsc.load_gather(
                    rp, [row, poff + (c * 16 + iota)])
            return carry3

        row0 = tok0 + j * _C

        def emit(s_const):
            rc, rr, rp = dsts[s_const]
            lax.fori_loop(
                0, _C,
                lambda i, c: assemble(i, c, rr, rp, asm[s_const]), 0)
            pltpu.make_async_copy(
                rc, out.at[pl.ds(row0, _C), pl.ds(0, 128)],
                semw[s_const]).start()
            pltpu.make_async_copy(
                asm[s_const], out.at[pl.ds(row0, _C), pl.ds(128, 192)],
                semw[s_const]).start()

        @pl.when(s == 0)
        def _():
            emit(0)

        @pl.when(s == 1)
        def _():
            emit(1)

        return carry

    lax.fori_loop(0, _STEPS, step, 0)
    wait_writes((_STEPS - 1) % 2)


def kernel(idx_char, idx_bound, idx_flag, idx_radical, idx_pinyin,
           W_char, W_bound, W_flag, W_radical, W_pinyin):
    idxs = [a.reshape(_N).astype(jnp.int32)
            for a in (idx_char, idx_bound, idx_flag, idx_radical, idx_pinyin)]
    tables = [W_char,
              W_bound.reshape(100 * 32),
              W_flag.reshape(100 * 32),
              W_radical.reshape(500, 128),
              W_pinyin.reshape(500, 128)]
    scratch = ([pltpu.VMEM((_PER_W,), jnp.int32) for _ in range(7)]
               + [pltpu.VMEM((100 * 32,), jnp.float32) for _ in range(2)]
               + [pltpu.VMEM((_C, 128), jnp.float32) for _ in range(6)]
               + [pltpu.VMEM((_C, 192), jnp.float32) for _ in range(2)]
               + [pltpu.SemaphoreType.DMA for _ in range(4)])
    k = pl.kernel(
        _sc_body,
        out_type=jax.ShapeDtypeStruct((_N, _TOT), jnp.float32),
        mesh=plsc.VectorSubcoreMesh(core_axis_name="c", subcore_axis_name="s"),
        scratch_types=scratch,
        compiler_params=pltpu.CompilerParams(needs_layout_passes=False),
    )
    out = k(*idxs, *tables)
    return out.reshape(_B, _L, _TOT)


# speedup vs baseline: 1.3918x; 1.2372x over previous
import jax
import jax.numpy as jnp
from jax import lax
from jax.experimental import pallas as pl
from jax.experimental.pallas import tpu as pltpu
from jax.experimental.pallas import tpu_sc as plsc

_TOT = 320
_B, _L = 4096, 50
_N = _B * _L
_C = 50
_NC, _NS = 2, 16
_NW = _NC * _NS
_PER_W = _N // _NW
_STEPS = _PER_W // _C


def _sc_body(ic, ib, if_, ir, ip, wc, wb, wf, wr, wp, out,
             ivc, ivb, ivf, ivr, ivp, tb, tf,
             q0c, q0r, q0p, q1c, q1r, q1p,
             c0, c1, r0, r1, p0, p1, a0, a1,
             sg0, sg1, sw0, sw1):
    wid = lax.axis_index("s") * _NC + lax.axis_index("c")
    tok0 = wid * _PER_W
    row_b0 = wid * _STEPS
    gsrc = (wc, wr, wp)
    qidx = ((q0c, q0r, q0p), (q1c, q1r, q1p))
    dsts = ((c0, r0, p0), (c1, r1, p1))
    asm = (a0, a1)
    semg = (sg0, sg1)
    semw = (sw0, sw1)

    for src, dst in ((ic, ivc), (ib, ivb), (if_, ivf), (ir, ivr), (ip, ivp)):
        pltpu.sync_copy(src.at[pl.ds(tok0, _PER_W)], dst)
    pltpu.sync_copy(wb, tb)
    pltpu.sync_copy(wf, tf)

    iota = lax.iota(jnp.int32, 16)
    tail_mask = iota < 2

    def repack(j, s):
        base = jnp.full((16,), j * _C, jnp.int32)
        for t, (flat, shift) in enumerate(((ivc, 0), (ivr, 1), (ivp, 1))):
            q = qidx[s][t]
            for c in range(3):
                v = plsc.load_gather(flat, [base + (c * 16 + iota)])
                q[pl.ds(c * 16, 16)] = lax.shift_right_logical(v, shift)
            v = plsc.load_gather(flat, [base + (48 + iota)])
            plsc.store_scatter(q, [48 + iota],
                               lax.shift_right_logical(v, shift),
                               mask=tail_mask)

    def fire_gathers(s):
        for t in range(3):
            pltpu.make_async_copy(
                gsrc[t].at[qidx[s][t]], dsts[s][t], semg[s]).start()

    def wait_gathers(s):
        for t in range(3):
            pltpu.make_async_copy(
                out.at[0, :, pl.ds(0, 128)], dsts[s][t], semg[s]).wait()

    def wait_writes(s):
        pltpu.make_async_copy(
            dsts[s][0], out.at[0, :, pl.ds(0, 128)], semw[s]).wait()
        pltpu.make_async_copy(
            asm[s], out.at[0, :, pl.ds(128, 192)], semw[s]).wait()

    repack(0, 0)
    fire_gathers(0)

    def step(j, carry):
        s = lax.rem(j, 2)

        @pl.when(s == 0)
        def _():
            wait_gathers(0)

        @pl.when(s == 1)
        def _():
            wait_gathers(1)

        @pl.when((j >= 1) & (s == 0))
        def _():
            wait_writes(1)

        @pl.when((j >= 1) & (s == 1))
        def _():
            wait_writes(0)

        @pl.when((j < _STEPS - 1) & (s == 0))
        def _():
            repack(j + 1, 1)
            fire_gathers(1)

        @pl.when((j < _STEPS - 1) & (s == 1))
        def _():
            repack(j + 1, 0)
            fire_gathers(0)

        def assemble(i, carry3, rr, rp, am):
            pos = jnp.full((16,), j * _C + i, jnp.int32)
            row = jnp.full((16,), i, jnp.int32)
            vb = plsc.load_gather(ivb, [pos]) * 32
            vf = plsc.load_gather(ivf, [pos]) * 32
            vr = plsc.load_gather(ivr, [pos])
            vp = plsc.load_gather(ivp, [pos])
            roff = (vr & 1) * 64
            poff = (vp & 1) * 64
            am[i, pl.ds(0, 16)] = plsc.load_gather(tb, [vb + iota])
            am[i, pl.ds(16, 16)] = plsc.load_gather(tb, [vb + (16 + iota)])
            am[i, pl.ds(32, 16)] = plsc.load_gather(tf, [vf + iota])
            am[i, pl.ds(48, 16)] = plsc.load_gather(tf, [vf + (16 + iota)])
            for c in range(4):
                am[i, pl.ds(64 + c * 16, 16)] = plsc.load_gather(
                    rr, [row, roff + (c * 16 + iota)])
            for c in range(4):
                am[i, pl.ds(128 + c * 16, 16)] = plsc.load_gather(
                    rp, [row, poff + (c * 16 + iota)])
            return carry3

        bb = row_b0 + j

        def emit(s_const):
            rc, rr, rp = dsts[s_const]
            lax.fori_loop(
                0, _C,
                lambda i, c: assemble(i, c, rr, rp, asm[s_const]), 0)
            pltpu.make_async_copy(
                rc, out.at[bb, :, pl.ds(0, 128)], semw[s_const]).start()
            pltpu.make_async_copy(
                asm[s_const], out.at[bb, :, pl.ds(128, 192)],
                semw[s_const]).start()

        @pl.when(s == 0)
        def _():
            emit(0)

        @pl.when(s == 1)
        def _():
            emit(1)

        return carry

    lax.fori_loop(0, _STEPS, step, 0)
    wait_writes((_STEPS - 1) % 2)


def kernel(idx_char, idx_bound, idx_flag, idx_radical, idx_pinyin,
           W_char, W_bound, W_flag, W_radical, W_pinyin):
    idxs = [a.reshape(_N).astype(jnp.int32)
            for a in (idx_char, idx_bound, idx_flag, idx_radical, idx_pinyin)]
    tables = [W_char,
              W_bound.reshape(100 * 32),
              W_flag.reshape(100 * 32),
              W_radical.reshape(500, 128),
              W_pinyin.reshape(500, 128)]
    scratch = ([pltpu.VMEM((_PER_W,), jnp.int32) for _ in range(5)]
               + [pltpu.VMEM((100 * 32,), jnp.float32) for _ in range(2)]
               + [pltpu.VMEM((_C,), jnp.int32) for _ in range(6)]
               + [pltpu.VMEM((_C, 128), jnp.float32) for _ in range(6)]
               + [pltpu.VMEM((_C, 192), jnp.float32) for _ in range(2)]
               + [pltpu.SemaphoreType.DMA for _ in range(4)])
    k = pl.kernel(
        _sc_body,
        out_type=jax.ShapeDtypeStruct((_B, _L, _TOT), jnp.float32),
        mesh=plsc.VectorSubcoreMesh(core_axis_name="c", subcore_axis_name="s"),
        scratch_types=scratch,
        compiler_params=pltpu.CompilerParams(needs_layout_passes=False),
    )
    return k(*idxs, *tables)
